# baseline (speedup 1.0000x reference)
import functools

import jax
import jax.numpy as jnp
from jax import lax
from jax.experimental import pallas as pl
from jax.experimental.pallas import tpu as pltpu
from jax.experimental.pallas import tpu_sc as plsc

GRID_N = 16384
INPUT_DIM = 512
OUTPUT_DIM = 100
BATCH = 1024

N_CHUNK = 4096
C_PAD = 128


def _bmu_body(x_ref, w_ref, bmu_ref, minval_ref, minarg_ref):
    j = pl.program_id(0)

    @pl.when(j == 0)
    def _init():
        minval_ref[...] = jnp.full_like(minval_ref, jnp.inf)
        minarg_ref[...] = jnp.zeros_like(minarg_ref)

    xb = x_ref[...]
    wb = w_ref[...]
    w2 = jnp.sum(wb * wb, axis=1, keepdims=True)
    wx = lax.dot_general(wb, xb, (((1,), (1,)), ((), ())),
                         preferred_element_type=jnp.float32)
    scores = w2 - 2.0 * wx
    local_min = jnp.min(scores, axis=0, keepdims=True)
    row = lax.broadcasted_iota(jnp.int32, scores.shape, 0) + j * N_CHUNK
    local_arg = jnp.min(
        jnp.where(scores == local_min, row, jnp.int32(2**30)),
        axis=0, keepdims=True)
    better = local_min < minval_ref[...]
    minval_ref[...] = jnp.where(better, local_min, minval_ref[...])
    minarg_ref[...] = jnp.where(better, local_arg, minarg_ref[...])

    @pl.when(j == pl.num_programs(0) - 1)
    def _emit():
        bmu_ref[...] = minarg_ref[...]


def _bmu(x, weights, *, interpret=False):
    nb = GRID_N // N_CHUNK
    return pl.pallas_call(
        _bmu_body,
        grid=(nb,),
        in_specs=[
            pl.BlockSpec((BATCH, INPUT_DIM), lambda j: (0, 0)),
            pl.BlockSpec((N_CHUNK, INPUT_DIM), lambda j: (j, 0)),
        ],
        out_specs=pl.BlockSpec((1, BATCH), lambda j: (0, 0)),
        out_shape=jax.ShapeDtypeStruct((1, BATCH), jnp.int32),
        scratch_shapes=[
            pltpu.VMEM((1, BATCH), jnp.float32),
            pltpu.VMEM((1, BATCH), jnp.int32),
        ],
        interpret=interpret,
    )(x, weights)


def _make_sc_gather():
    info = plsc.get_sparse_core_info()
    nc, ns = info.num_cores, info.num_subcores
    nw = nc * ns
    b_per_w = BATCH // nw
    mesh = plsc.VectorSubcoreMesh(core_axis_name="c", subcore_axis_name="s")

    @functools.partial(
        pl.kernel,
        mesh=mesh,
        out_type=jax.ShapeDtypeStruct((BATCH, C_PAD), jnp.float32),
        scratch_types=[
            pltpu.VMEM((b_per_w,), jnp.int32),
            pltpu.VMEM((b_per_w, C_PAD), jnp.float32),
            pltpu.SemaphoreType.DMA,
        ],
    )
    def gather_k(labels_hbm, idx_hbm, out_hbm, idx_v, rows_v, sem):
        wid = lax.axis_index("s") * nc + lax.axis_index("c")
        base = wid * b_per_w
        pltpu.sync_copy(idx_hbm.at[pl.ds(base, b_per_w)], idx_v)
        pltpu.async_copy(labels_hbm.at[idx_v], rows_v, sem).wait()
        pltpu.sync_copy(rows_v, out_hbm.at[pl.ds(base, b_per_w)])

    return gather_k


def _softmax_body(g_ref, o_ref):
    g = g_ref[...]
    m = jnp.max(g, axis=1, keepdims=True)
    e = jnp.exp(g - m)
    o_ref[...] = (e / jnp.sum(e, axis=1, keepdims=True))[:, :OUTPUT_DIM]


def _softmax(g, *, interpret=False):
    return pl.pallas_call(
        _softmax_body,
        out_shape=jax.ShapeDtypeStruct((BATCH, OUTPUT_DIM), jnp.float32),
        interpret=interpret,
    )(g)


def kernel(x, weights, labels):
    bmu = _bmu(x, weights)[0, :]
    labels_p = jnp.pad(labels, ((0, 0), (0, C_PAD - OUTPUT_DIM)),
                       constant_values=-1e30)
    gathered = _make_sc_gather()(labels_p, bmu)
    return _softmax(gathered)


# speedup vs baseline: 1.0042x; 1.0042x over previous
import functools

import jax
import jax.numpy as jnp
from jax import lax
from jax.experimental import pallas as pl
from jax.experimental.pallas import tpu as pltpu
from jax.experimental.pallas import tpu_sc as plsc

GRID_N = 16384
INPUT_DIM = 512
OUTPUT_DIM = 100
BATCH = 1024

N_CHUNK = 2048
C_PAD = 128


def _bmu_body(x_ref, w_ref, bmu_ref, minval_ref, minarg_ref):
    j = pl.program_id(0)

    @pl.when(j == 0)
    def _init():
        minval_ref[...] = jnp.full_like(minval_ref, jnp.inf)
        minarg_ref[...] = jnp.zeros_like(minarg_ref)

    xb = x_ref[...]
    wb = w_ref[...]
    w2 = jnp.sum(wb * wb, axis=1, keepdims=True)
    wx = lax.dot_general(wb, xb, (((1,), (1,)), ((), ())),
                         preferred_element_type=jnp.float32)
    scores = w2 - 2.0 * wx
    local_min = jnp.min(scores, axis=0, keepdims=True)
    row = lax.broadcasted_iota(jnp.int32, scores.shape, 0) + j * N_CHUNK
    local_arg = jnp.min(
        jnp.where(scores == local_min, row, jnp.int32(2**30)),
        axis=0, keepdims=True)
    better = local_min < minval_ref[...]
    minval_ref[...] = jnp.where(better, local_min, minval_ref[...])
    minarg_ref[...] = jnp.where(better, local_arg, minarg_ref[...])

    @pl.when(j == pl.num_programs(0) - 1)
    def _emit():
        bmu_ref[...] = minarg_ref[...]


def _bmu(x, weights, *, interpret=False):
    nb = GRID_N // N_CHUNK
    return pl.pallas_call(
        _bmu_body,
        grid=(nb,),
        in_specs=[
            pl.BlockSpec((BATCH, INPUT_DIM), lambda j: (0, 0)),
            pl.BlockSpec((N_CHUNK, INPUT_DIM), lambda j: (j, 0)),
        ],
        out_specs=pl.BlockSpec((1, BATCH), lambda j: (0, 0)),
        out_shape=jax.ShapeDtypeStruct((1, BATCH), jnp.int32),
        scratch_shapes=[
            pltpu.VMEM((1, BATCH), jnp.float32),
            pltpu.VMEM((1, BATCH), jnp.int32),
        ],
        interpret=interpret,
    )(x, weights)


def _make_sc_gather():
    info = plsc.get_sparse_core_info()
    nc, ns = info.num_cores, info.num_subcores
    nw = nc * ns
    b_per_w = BATCH // nw
    mesh = plsc.VectorSubcoreMesh(core_axis_name="c", subcore_axis_name="s")

    @functools.partial(
        pl.kernel,
        mesh=mesh,
        out_type=jax.ShapeDtypeStruct((BATCH, C_PAD), jnp.float32),
        scratch_types=[
            pltpu.VMEM((b_per_w,), jnp.int32),
            pltpu.VMEM((b_per_w, C_PAD), jnp.float32),
            pltpu.SemaphoreType.DMA,
        ],
    )
    def gather_k(labels_hbm, idx_hbm, out_hbm, idx_v, rows_v, sem):
        wid = lax.axis_index("s") * nc + lax.axis_index("c")
        base = wid * b_per_w
        pltpu.sync_copy(idx_hbm.at[pl.ds(base, b_per_w)], idx_v)
        pltpu.async_copy(labels_hbm.at[idx_v], rows_v, sem).wait()
        pltpu.sync_copy(rows_v, out_hbm.at[pl.ds(base, b_per_w)])

    return gather_k


def _softmax_body(g_ref, o_ref):
    g = g_ref[...]
    m = jnp.max(g, axis=1, keepdims=True)
    e = jnp.exp(g - m)
    o_ref[...] = (e / jnp.sum(e, axis=1, keepdims=True))[:, :OUTPUT_DIM]


def _softmax(g, *, interpret=False):
    return pl.pallas_call(
        _softmax_body,
        out_shape=jax.ShapeDtypeStruct((BATCH, OUTPUT_DIM), jnp.float32),
        interpret=interpret,
    )(g)


def kernel(x, weights, labels):
    bmu = _bmu(x, weights)[0, :]
    labels_p = jnp.pad(labels, ((0, 0), (0, C_PAD - OUTPUT_DIM)),
                       constant_values=-1e30)
    gathered = _make_sc_gather()(labels_p, bmu)
    return _softmax(gathered)


# speedup vs baseline: 1.0061x; 1.0019x over previous
import functools

import jax
import jax.numpy as jnp
from jax import lax
from jax.experimental import pallas as pl
from jax.experimental.pallas import tpu as pltpu
from jax.experimental.pallas import tpu_sc as plsc

GRID_N = 16384
INPUT_DIM = 512
OUTPUT_DIM = 100
BATCH = 1024

N_CHUNK = 2048
C_PAD = 128


def _bmu_body(x_ref, w_ref, bmu_ref, minval_ref, minarg_ref):
    j = pl.program_id(0)

    @pl.when(j == 0)
    def _init():
        minval_ref[...] = jnp.full_like(minval_ref, jnp.inf)
        minarg_ref[...] = jnp.zeros_like(minarg_ref)

    xb = x_ref[...]
    wb = w_ref[...]
    w2 = jnp.sum(wb * wb, axis=1, keepdims=True)
    wx = lax.dot_general(wb, xb, (((1,), (1,)), ((), ())),
                         preferred_element_type=jnp.float32)
    scores = w2 - 2.0 * wx
    local_min = jnp.min(scores, axis=0, keepdims=True)
    row = lax.broadcasted_iota(jnp.int32, scores.shape, 0) + j * N_CHUNK
    local_arg = jnp.min(
        jnp.where(scores == local_min, row, jnp.int32(2**30)),
        axis=0, keepdims=True)
    better = local_min < minval_ref[...]
    minval_ref[...] = jnp.where(better, local_min, minval_ref[...])
    minarg_ref[...] = jnp.where(better, local_arg, minarg_ref[...])

    @pl.when(j == pl.num_programs(0) - 1)
    def _emit():
        bmu_ref[...] = minarg_ref[...]


def _bmu(x, weights, *, interpret=False):
    nb = GRID_N // N_CHUNK
    return pl.pallas_call(
        _bmu_body,
        grid=(nb,),
        in_specs=[
            pl.BlockSpec((BATCH, INPUT_DIM), lambda j: (0, 0)),
            pl.BlockSpec((N_CHUNK, INPUT_DIM), lambda j: (j, 0)),
        ],
        out_specs=pl.BlockSpec((1, BATCH), lambda j: (0, 0)),
        out_shape=jax.ShapeDtypeStruct((1, BATCH), jnp.int32),
        scratch_shapes=[
            pltpu.VMEM((1, BATCH), jnp.float32),
            pltpu.VMEM((1, BATCH), jnp.int32),
        ],
        interpret=interpret,
    )(x, weights)


def _make_sc_gather():
    info = plsc.get_sparse_core_info()
    nc, ns = info.num_cores, info.num_subcores
    nw = nc * ns
    b_per_w = BATCH // nw
    mesh = plsc.VectorSubcoreMesh(core_axis_name="c", subcore_axis_name="s")

    @functools.partial(
        pl.kernel,
        mesh=mesh,
        out_type=jax.ShapeDtypeStruct((BATCH, C_PAD), jnp.float32),
        scratch_types=[
            pltpu.VMEM((b_per_w,), jnp.int32),
            pltpu.VMEM((b_per_w, C_PAD), jnp.float32),
            pltpu.SemaphoreType.DMA,
        ],
    )
    def gather_k(labels_hbm, idx_hbm, out_hbm, idx_v, rows_v, sem):
        wid = lax.axis_index("s") * nc + lax.axis_index("c")
        base = wid * b_per_w
        pltpu.sync_copy(idx_hbm.at[pl.ds(base, b_per_w)], idx_v)
        pltpu.async_copy(labels_hbm.at[idx_v], rows_v, sem).wait()
        pltpu.sync_copy(rows_v, out_hbm.at[pl.ds(base, b_per_w)])

    return gather_k


def _softmax_body(g_ref, o_ref):
    g = g_ref[...]
    m = jnp.max(g, axis=1, keepdims=True)
    e = jnp.exp(g - m)
    o_ref[...] = (e / jnp.sum(e, axis=1, keepdims=True))[:, :OUTPUT_DIM]


def _softmax(g, *, interpret=False):
    return pl.pallas_call(
        _softmax_body,
        out_shape=jax.ShapeDtypeStruct((BATCH, OUTPUT_DIM), jnp.float32),
        interpret=interpret,
    )(g)


def kernel(x, weights, labels):
    bmu = _bmu(x, weights)[0, :]
    labels_p = jnp.concatenate(
        [labels, jnp.full((GRID_N, C_PAD - OUTPUT_DIM), -1e30, jnp.float32)],
        axis=1)
    gathered = _make_sc_gather()(labels_p, bmu)
    return _softmax(gathered)


# speedup vs baseline: 1.0129x; 1.0068x over previous
import functools

import jax
import jax.numpy as jnp
from jax import lax
from jax.experimental import pallas as pl
from jax.experimental.pallas import tpu as pltpu
from jax.experimental.pallas import tpu_sc as plsc

GRID_N = 16384
INPUT_DIM = 512
OUTPUT_DIM = 100
BATCH = 1024

N_CHUNK = 2048
C_PAD = 128


def _bmu_body(x_ref, w_ref, bmu_ref, minval_ref, minarg_ref):
    j = pl.program_id(0)

    @pl.when(j == 0)
    def _init():
        minval_ref[...] = jnp.full_like(minval_ref, jnp.inf)
        minarg_ref[...] = jnp.zeros_like(minarg_ref)

    xb = x_ref[...]
    wb = w_ref[...]
    w2 = jnp.sum(wb * wb, axis=1, keepdims=True)
    wx = lax.dot_general(wb, xb, (((1,), (1,)), ((), ())),
                         preferred_element_type=jnp.float32)
    scores = w2 - 2.0 * wx
    local_min = jnp.min(scores, axis=0, keepdims=True)
    row = lax.broadcasted_iota(jnp.int32, scores.shape, 0) + j * N_CHUNK
    local_arg = jnp.min(
        jnp.where(scores == local_min, row, jnp.int32(2**30)),
        axis=0, keepdims=True)
    better = local_min < minval_ref[...]
    minval_ref[...] = jnp.where(better, local_min, minval_ref[...])
    minarg_ref[...] = jnp.where(better, local_arg, minarg_ref[...])

    @pl.when(j == pl.num_programs(0) - 1)
    def _emit():
        bmu_ref[...] = minarg_ref[...]


def _bmu(x, weights, *, interpret=False):
    nb = GRID_N // N_CHUNK
    return pl.pallas_call(
        _bmu_body,
        grid=(nb,),
        in_specs=[
            pl.BlockSpec((BATCH, INPUT_DIM), lambda j: (0, 0)),
            pl.BlockSpec((N_CHUNK, INPUT_DIM), lambda j: (j, 0)),
        ],
        out_specs=pl.BlockSpec((1, BATCH), lambda j: (0, 0)),
        out_shape=jax.ShapeDtypeStruct((1, BATCH), jnp.int32),
        scratch_shapes=[
            pltpu.VMEM((1, BATCH), jnp.float32),
            pltpu.VMEM((1, BATCH), jnp.int32),
        ],
        interpret=interpret,
    )(x, weights)


def _make_sc_gather():
    info = plsc.get_sparse_core_info()
    nc, ns = info.num_cores, info.num_subcores
    nw = nc * ns
    b_per_w = BATCH // nw
    mesh = plsc.VectorSubcoreMesh(core_axis_name="c", subcore_axis_name="s")
    nchunk = C_PAD // 16

    @functools.partial(
        pl.kernel,
        mesh=mesh,
        out_type=jax.ShapeDtypeStruct((BATCH, C_PAD), jnp.float32),
        scratch_types=[
            pltpu.VMEM((b_per_w,), jnp.int32),
            pltpu.VMEM((b_per_w, C_PAD), jnp.float32),
            pltpu.SemaphoreType.DMA,
        ],
        compiler_params=pltpu.CompilerParams(needs_layout_passes=False),
    )
    def gather_k(labels_hbm, idx_hbm, out_hbm, idx_v, rows_v, sem):
        wid = lax.axis_index("s") * nc + lax.axis_index("c")
        base = wid * b_per_w
        pltpu.sync_copy(idx_hbm.at[pl.ds(base, b_per_w)], idx_v)
        pltpu.async_copy(labels_hbm.at[idx_v], rows_v, sem).wait()

        def srow(i, carry):
            v = [rows_v[i, pl.ds(16 * k, 16)] for k in range(nchunk)]
            t = v[0]
            for k in range(1, nchunk):
                t = jnp.maximum(t, v[k])
            m = jnp.max(t)
            e = [jnp.exp(vk - m) for vk in v]
            t = e[0]
            for k in range(1, nchunk):
                t = t + e[k]
            s = jnp.broadcast_to(jnp.sum(t), (16,))
            for k in range(nchunk):
                rows_v[i, pl.ds(16 * k, 16)] = e[k] / s
            return carry

        lax.fori_loop(0, b_per_w, srow, 0)
        pltpu.sync_copy(rows_v, out_hbm.at[pl.ds(base, b_per_w)])

    return gather_k


def _softmax_body(g_ref, o_ref):
    g = g_ref[...]
    m = jnp.max(g, axis=1, keepdims=True)
    e = jnp.exp(g - m)
    o_ref[...] = (e / jnp.sum(e, axis=1, keepdims=True))[:, :OUTPUT_DIM]


def _softmax(g, *, interpret=False):
    return pl.pallas_call(
        _softmax_body,
        out_shape=jax.ShapeDtypeStruct((BATCH, OUTPUT_DIM), jnp.float32),
        interpret=interpret,
    )(g)


def kernel(x, weights, labels):
    bmu = _bmu(x, weights)[0, :]
    labels_p = jnp.concatenate(
        [labels, jnp.full((GRID_N, C_PAD - OUTPUT_DIM), -1e30, jnp.float32)],
        axis=1)
    out_p = _make_sc_gather()(labels_p, bmu)
    return out_p[:, :OUTPUT_DIM]


# speedup vs baseline: 1.0154x; 1.0024x over previous
import functools

import jax
import jax.numpy as jnp
from jax import lax
from jax.experimental import pallas as pl
from jax.experimental.pallas import tpu as pltpu
from jax.experimental.pallas import tpu_sc as plsc

GRID_N = 16384
INPUT_DIM = 512
OUTPUT_DIM = 100
BATCH = 1024

N_CHUNK = 2048
C_PAD = 128


def _bmu_body(x_ref, w_ref, bmu_ref, minval_ref, minarg_ref):
    j = pl.program_id(0)

    @pl.when(j == 0)
    def _init():
        minval_ref[...] = jnp.full_like(minval_ref, jnp.inf)
        minarg_ref[...] = jnp.zeros_like(minarg_ref)

    xb = x_ref[...]
    wb = w_ref[...]
    w2 = jnp.sum(wb * wb, axis=1, keepdims=True)
    wx = lax.dot_general(wb, xb, (((1,), (1,)), ((), ())),
                         preferred_element_type=jnp.float32)
    scores = w2 - 2.0 * wx
    local_min = jnp.min(scores, axis=0, keepdims=True)
    row = lax.broadcasted_iota(jnp.int32, scores.shape, 0) + j * N_CHUNK
    local_arg = jnp.min(
        jnp.where(scores == local_min, row, jnp.int32(2**30)),
        axis=0, keepdims=True)
    better = local_min < minval_ref[...]
    minval_ref[...] = jnp.where(better, local_min, minval_ref[...])
    minarg_ref[...] = jnp.where(better, local_arg, minarg_ref[...])

    @pl.when(j == pl.num_programs(0) - 1)
    def _emit():
        bmu_ref[...] = minarg_ref[...]


def _bmu(x, weights, *, interpret=False):
    nb = GRID_N // N_CHUNK
    return pl.pallas_call(
        _bmu_body,
        grid=(nb,),
        in_specs=[
            pl.BlockSpec((BATCH, INPUT_DIM), lambda j: (0, 0)),
            pl.BlockSpec((N_CHUNK, INPUT_DIM), lambda j: (j, 0)),
        ],
        out_specs=pl.BlockSpec((1, BATCH), lambda j: (0, 0)),
        out_shape=jax.ShapeDtypeStruct((1, BATCH), jnp.int32),
        scratch_shapes=[
            pltpu.VMEM((1, BATCH), jnp.float32),
            pltpu.VMEM((1, BATCH), jnp.int32),
        ],
        interpret=interpret,
    )(x, weights)


def _make_sc_gather():
    info = plsc.get_sparse_core_info()
    nc, ns = info.num_cores, info.num_subcores
    nw = nc * ns
    b_per_w = BATCH // nw
    mesh = plsc.VectorSubcoreMesh(core_axis_name="c", subcore_axis_name="s")
    nchunk = C_PAD // 16

    @functools.partial(
        pl.kernel,
        mesh=mesh,
        out_type=jax.ShapeDtypeStruct((BATCH, C_PAD), jnp.float32),
        scratch_types=[
            pltpu.VMEM((b_per_w,), jnp.int32),
            pltpu.VMEM((b_per_w, C_PAD), jnp.float32),
            pltpu.SemaphoreType.DMA,
        ],
        compiler_params=pltpu.CompilerParams(needs_layout_passes=False),
    )
    def gather_k(labels_hbm, idx_hbm, out_hbm, idx_v, rows_v, sem):
        wid = lax.axis_index("s") * nc + lax.axis_index("c")
        base = wid * b_per_w
        pltpu.sync_copy(idx_hbm.at[pl.ds(base, b_per_w)], idx_v)
        pltpu.async_copy(labels_hbm.at[idx_v], rows_v, sem).wait()

        def srow(i, carry):
            v = [rows_v[i, pl.ds(16 * k, 16)] for k in range(nchunk)]
            t = v[0]
            for k in range(1, nchunk):
                t = jnp.maximum(t, v[k])
            m = jnp.max(t)
            e = [jnp.exp(vk - m) for vk in v]
            t = e[0]
            for k in range(1, nchunk):
                t = t + e[k]
            s = jnp.broadcast_to(jnp.sum(t), (16,))
            for k in range(nchunk):
                rows_v[i, pl.ds(16 * k, 16)] = e[k] / s
            return carry

        lax.fori_loop(0, b_per_w, srow, 0)
        pltpu.sync_copy(rows_v, out_hbm.at[pl.ds(base, b_per_w)])

    return gather_k


def kernel(x, weights, labels):
    bmu = _bmu(x, weights)[0, :]
    labels_p = jnp.concatenate(
        [labels, jnp.full((GRID_N, C_PAD - OUTPUT_DIM), -1e30, jnp.float32)],
        axis=1)
    out_p = _make_sc_gather()(labels_p, bmu)
    return out_p[:, :OUTPUT_DIM]
